# baseline (speedup 1.0000x reference)
import jax
import jax.numpy as jnp
from jax import lax
from jax.experimental import pallas as pl
from jax.experimental.pallas import tpu as pltpu
from jax.experimental.pallas import tpu_sc as plsc

_W = 128
_NC = 2
_NS = 16
_NW = _NC * _NS
_NB = 4


def kernel(x, table):
    n = x.shape[0]
    d = table.shape[1]
    idx = x.reshape(n).astype(jnp.int32)

    n_full = n // _W
    rem = n - n_full * _W
    n_win = n_full + (1 if rem else 0)
    wins_per_w = -(-n_win // _NW)
    per_w = wins_per_w * _W
    last_cnt = n - per_w * (_NW - 1)
    supersteps = -(-wins_per_w // _NB)
    assert wins_per_w >= _NB and n_full - wins_per_w * (_NW - 1) >= _NB
    if rem:
        assert n_full // wins_per_w == _NW - 1 and rem % 8 == 0
    rem_rows = rem if rem else 8

    mesh = plsc.VectorSubcoreMesh(core_axis_name="c", subcore_axis_name="s")

    v = table.shape[0]
    row_ids = jnp.arange(per_w * _NW, dtype=jnp.int32).reshape(
        _NW, wins_per_w, _W)
    scratch = (
        [pltpu.VMEM_SHARED((v, d), table.dtype)]
        + [pltpu.VMEM((per_w,), jnp.int32)]
        + [pltpu.VMEM((wins_per_w, _W), jnp.int32)]
        + [pltpu.VMEM((_W, d), table.dtype) for _ in range(_NB)]
        + [pltpu.VMEM((rem_rows, d), table.dtype)]
        + [pltpu.SemaphoreType.DMA for _ in range(2 * _NB)]
    )

    @pl.kernel(
        out_type=jax.ShapeDtypeStruct((n, d), table.dtype),
        mesh=mesh,
        scratch_types=scratch,
    )
    def gather_kernel(table_hbm, idx_hbm, rid_hbm, out_hbm, table_sp, idx_v,
                      rid_v, *rest):
        bufs = rest[:_NB]
        rem_buf = rest[_NB]
        gsems = rest[_NB + 1:2 * _NB + 1]
        wsems = rest[2 * _NB + 1:]
        w = lax.axis_index("s") * _NC + lax.axis_index("c")
        full_mine = jnp.clip(n_full - wins_per_w * w, 0, wins_per_w)

        @pl.when(w < _NW - 1)
        def _():
            pltpu.async_copy(idx_hbm.at[pl.ds(w * per_w, per_w)], idx_v,
                             gsems[0])

        @pl.when(w == _NW - 1)
        def _():
            pltpu.async_copy(
                idx_hbm.at[pl.ds((_NW - 1) * per_w, last_cnt)],
                idx_v.at[pl.ds(0, last_cnt)],
                gsems[0],
            )

        pltpu.async_copy(rid_hbm.at[w], rid_v, gsems[1])

        @pl.when(lax.axis_index("s") == 0)
        def _():
            pltpu.sync_copy(table_hbm, table_sp)

        pltpu.make_async_copy(rid_hbm.at[0], rid_v, gsems[1]).wait()

        @pl.when(w < _NW - 1)
        def _():
            pltpu.make_async_copy(
                idx_hbm.at[pl.ds(0, per_w)], idx_v, gsems[0]).wait()

        @pl.when(w == _NW - 1)
        def _():
            pltpu.make_async_copy(
                idx_hbm.at[pl.ds(0, last_cnt)],
                idx_v.at[pl.ds(0, last_cnt)], gsems[0]).wait()

        plsc.subcore_barrier()

        def issue_gather(j, buf, sem):
            pltpu.async_copy(
                table_sp.at[idx_v.at[pl.ds(j * _W, _W)]], buf, sem)

        def wait_gather(buf, sem):
            pltpu.make_async_copy(out_hbm.at[pl.ds(0, _W)], buf, sem).wait()

        def issue_write(j, buf, sem):
            pltpu.async_copy(buf, out_hbm.at[rid_v.at[j]], sem)

        def wait_write(buf, sem):
            pltpu.make_async_copy(buf, out_hbm.at[pl.ds(0, _W)], sem).wait()

        for i in range(_NB):
            issue_gather(i, bufs[i], gsems[i])
        for i in range(_NB):
            wait_gather(bufs[i], gsems[i])
            issue_write(i, bufs[i], wsems[i])

        @pl.loop(1, supersteps)
        def _(t):
            j_base = _NB * t
            for i in range(_NB):
                @pl.when(j_base + i < full_mine)
                def _(i=i):
                    wait_write(bufs[i], wsems[i])
                    issue_gather(j_base + i, bufs[i], gsems[i])
            for i in range(_NB):
                @pl.when(j_base + i < full_mine)
                def _(i=i):
                    wait_gather(bufs[i], gsems[i])
                    issue_write(j_base + i, bufs[i], wsems[i])

        for i in range(_NB):
            wait_write(bufs[i], wsems[i])

        if rem:
            @pl.when(w == _NW - 1)
            def _():
                lo = (n_full - wins_per_w * (_NW - 1)) * _W
                pltpu.async_copy(
                    table_sp.at[idx_v.at[pl.ds(lo, rem)]], rem_buf,
                    gsems[0]).wait()
                pltpu.sync_copy(
                    rem_buf, out_hbm.at[pl.ds(n_full * _W, rem)])

    return gather_kernel(table, idx, row_ids)


# speedup vs baseline: 1.0232x; 1.0232x over previous
import jax
import jax.numpy as jnp
from jax import lax
from jax.experimental import pallas as pl
from jax.experimental.pallas import tpu as pltpu
from jax.experimental.pallas import tpu_sc as plsc

_W = 128
_NC = 2
_NS = 16
_NW = _NC * _NS
_NB = 4


def kernel(x, table):
    n = x.shape[0]
    d = table.shape[1]
    idx = x.reshape(n).astype(jnp.int32)

    n_full = n // _W
    rem = n - n_full * _W
    n_win = n_full + (1 if rem else 0)
    wins_per_w = -(-n_win // _NW)
    per_w = wins_per_w * _W
    last_cnt = n - per_w * (_NW - 1)
    supersteps = -(-wins_per_w // _NB)
    assert wins_per_w >= _NB and n_full - wins_per_w * (_NW - 1) >= _NB
    if rem:
        assert n_full // wins_per_w == _NW - 1 and rem % 8 == 0
    rem_rows = rem if rem else 8

    mesh = plsc.VectorSubcoreMesh(core_axis_name="c", subcore_axis_name="s")

    v = table.shape[0]
    scratch = (
        [pltpu.VMEM_SHARED((v, d), table.dtype)]
        + [pltpu.VMEM((per_w,), jnp.int32)]
        + [pltpu.VMEM((_W, d), table.dtype) for _ in range(_NB)]
        + [pltpu.VMEM((rem_rows, d), table.dtype)]
        + [pltpu.SemaphoreType.DMA for _ in range(2 * _NB)]
    )

    @pl.kernel(
        out_type=jax.ShapeDtypeStruct((n, d), table.dtype),
        mesh=mesh,
        scratch_types=scratch,
    )
    def gather_kernel(table_hbm, idx_hbm, out_hbm, table_sp, idx_v, *rest):
        bufs = rest[:_NB]
        rem_buf = rest[_NB]
        gsems = rest[_NB + 1:2 * _NB + 1]
        wsems = rest[2 * _NB + 1:]
        w = lax.axis_index("s") * _NC + lax.axis_index("c")
        full_mine = jnp.clip(n_full - wins_per_w * w, 0, wins_per_w)

        @pl.when(w < _NW - 1)
        def _():
            pltpu.async_copy(idx_hbm.at[pl.ds(w * per_w, per_w)], idx_v,
                             gsems[0])

        @pl.when(w == _NW - 1)
        def _():
            pltpu.async_copy(
                idx_hbm.at[pl.ds((_NW - 1) * per_w, last_cnt)],
                idx_v.at[pl.ds(0, last_cnt)],
                gsems[0],
            )

        @pl.when(lax.axis_index("s") == 0)
        def _():
            pltpu.sync_copy(table_hbm, table_sp)

        @pl.when(w < _NW - 1)
        def _():
            pltpu.make_async_copy(
                idx_hbm.at[pl.ds(0, per_w)], idx_v, gsems[0]).wait()

        @pl.when(w == _NW - 1)
        def _():
            pltpu.make_async_copy(
                idx_hbm.at[pl.ds(0, last_cnt)],
                idx_v.at[pl.ds(0, last_cnt)], gsems[0]).wait()

        plsc.subcore_barrier()

        def issue_gather(j, buf, sem):
            pltpu.async_copy(
                table_sp.at[idx_v.at[pl.ds(j * _W, _W)]], buf, sem)

        def wait_gather(buf, sem):
            pltpu.make_async_copy(out_hbm.at[pl.ds(0, _W)], buf, sem).wait()

        def issue_write(j, buf, sem):
            g = w * wins_per_w + j
            pltpu.async_copy(buf, out_hbm.at[pl.ds(g * _W, _W)], sem)

        def wait_write(buf, sem):
            pltpu.make_async_copy(buf, out_hbm.at[pl.ds(0, _W)], sem).wait()

        for i in range(_NB):
            issue_gather(i, bufs[i], gsems[i])
        for i in range(_NB):
            wait_gather(bufs[i], gsems[i])
            issue_write(i, bufs[i], wsems[i])

        @pl.loop(1, supersteps)
        def _(t):
            j_base = _NB * t
            for i in range(_NB):
                @pl.when(j_base + i < full_mine)
                def _(i=i):
                    wait_write(bufs[i], wsems[i])
                    issue_gather(j_base + i, bufs[i], gsems[i])
            for i in range(_NB):
                @pl.when(j_base + i < full_mine)
                def _(i=i):
                    wait_gather(bufs[i], gsems[i])
                    issue_write(j_base + i, bufs[i], wsems[i])

        for i in range(_NB):
            wait_write(bufs[i], wsems[i])

        if rem:
            @pl.when(w == _NW - 1)
            def _():
                lo = (n_full - wins_per_w * (_NW - 1)) * _W
                pltpu.async_copy(
                    table_sp.at[idx_v.at[pl.ds(lo, rem)]], rem_buf,
                    gsems[0]).wait()
                pltpu.sync_copy(
                    rem_buf, out_hbm.at[pl.ds(n_full * _W, rem)])

    return gather_kernel(table, idx)


# speedup vs baseline: 1.0237x; 1.0005x over previous
import jax
import jax.numpy as jnp
from jax import lax
from jax.experimental import pallas as pl
from jax.experimental.pallas import tpu as pltpu
from jax.experimental.pallas import tpu_sc as plsc

_W = 128
_NC = 2
_NS = 16
_NW = _NC * _NS
_NB = 4


def kernel(x, table):
    n = x.shape[0]
    d = table.shape[1]
    idx = x.reshape(n).astype(jnp.int32)

    n_full = n // _W
    rem = n - n_full * _W
    n_win = n_full + (1 if rem else 0)
    base_wins = n_win // _NW
    extra = n_win % _NW
    wmax = base_wins + (1 if extra else 0)
    per_w = wmax * _W
    c_hi = (base_wins + 1) * _W
    c_lo = base_wins * _W
    last_base = extra + base_wins * (_NW - 1)
    last_cnt = n - last_base * _W
    supersteps = -(-wmax // _NB)
    assert base_wins - (1 if rem else 0) >= _NB
    if rem:
        assert rem % 8 == 0
    rem_rows = rem if rem else 8

    mesh = plsc.VectorSubcoreMesh(core_axis_name="c", subcore_axis_name="s")

    v = table.shape[0]
    scratch = (
        [pltpu.VMEM_SHARED((v, d), table.dtype)]
        + [pltpu.VMEM((per_w,), jnp.int32)]
        + [pltpu.VMEM((_W, d), table.dtype) for _ in range(_NB)]
        + [pltpu.VMEM((rem_rows, d), table.dtype)]
        + [pltpu.SemaphoreType.DMA for _ in range(2 * _NB)]
    )

    @pl.kernel(
        out_type=jax.ShapeDtypeStruct((n, d), table.dtype),
        mesh=mesh,
        scratch_types=scratch,
    )
    def gather_kernel(table_hbm, idx_hbm, out_hbm, table_sp, idx_v, *rest):
        bufs = rest[:_NB]
        rem_buf = rest[_NB]
        gsems = rest[_NB + 1:2 * _NB + 1]
        wsems = rest[2 * _NB + 1:]
        w = lax.axis_index("s") * _NC + lax.axis_index("c")
        base_mine = base_wins * w + jnp.minimum(w, extra)
        wins_mine = jnp.where(w < extra, base_wins + 1, base_wins)
        full_mine = wins_mine - (
            jnp.where(w == _NW - 1, 1, 0) if rem else 0)

        @pl.when(jnp.logical_and(w < extra, w < _NW - 1))
        def _():
            pltpu.async_copy(idx_hbm.at[pl.ds(base_mine * _W, c_hi)],
                             idx_v.at[pl.ds(0, c_hi)], gsems[0])

        @pl.when(jnp.logical_and(w >= extra, w < _NW - 1))
        def _():
            pltpu.async_copy(idx_hbm.at[pl.ds(base_mine * _W, c_lo)],
                             idx_v.at[pl.ds(0, c_lo)], gsems[0])

        @pl.when(w == _NW - 1)
        def _():
            pltpu.async_copy(
                idx_hbm.at[pl.ds(last_base * _W, last_cnt)],
                idx_v.at[pl.ds(0, last_cnt)],
                gsems[0],
            )

        @pl.when(lax.axis_index("s") == 0)
        def _():
            pltpu.sync_copy(table_hbm, table_sp)

        @pl.when(jnp.logical_and(w < extra, w < _NW - 1))
        def _():
            pltpu.make_async_copy(
                idx_hbm.at[pl.ds(0, c_hi)],
                idx_v.at[pl.ds(0, c_hi)], gsems[0]).wait()

        @pl.when(jnp.logical_and(w >= extra, w < _NW - 1))
        def _():
            pltpu.make_async_copy(
                idx_hbm.at[pl.ds(0, c_lo)],
                idx_v.at[pl.ds(0, c_lo)], gsems[0]).wait()

        @pl.when(w == _NW - 1)
        def _():
            pltpu.make_async_copy(
                idx_hbm.at[pl.ds(0, last_cnt)],
                idx_v.at[pl.ds(0, last_cnt)], gsems[0]).wait()

        plsc.subcore_barrier()

        def issue_gather(j, buf, sem):
            pltpu.async_copy(
                table_sp.at[idx_v.at[pl.ds(j * _W, _W)]], buf, sem)

        def wait_gather(buf, sem):
            pltpu.make_async_copy(out_hbm.at[pl.ds(0, _W)], buf, sem).wait()

        def issue_write(j, buf, sem):
            g = base_mine + j
            pltpu.async_copy(buf, out_hbm.at[pl.ds(g * _W, _W)], sem)

        def wait_write(buf, sem):
            pltpu.make_async_copy(buf, out_hbm.at[pl.ds(0, _W)], sem).wait()

        for i in range(_NB):
            issue_gather(i, bufs[i], gsems[i])
        for i in range(_NB):
            wait_gather(bufs[i], gsems[i])
            issue_write(i, bufs[i], wsems[i])

        @pl.loop(1, supersteps)
        def _(t):
            j_base = _NB * t
            for i in range(_NB):
                @pl.when(j_base + i < full_mine)
                def _(i=i):
                    wait_write(bufs[i], wsems[i])
                    issue_gather(j_base + i, bufs[i], gsems[i])
            for i in range(_NB):
                @pl.when(j_base + i < full_mine)
                def _(i=i):
                    wait_gather(bufs[i], gsems[i])
                    issue_write(j_base + i, bufs[i], wsems[i])

        for i in range(_NB):
            wait_write(bufs[i], wsems[i])

        if rem:
            @pl.when(w == _NW - 1)
            def _():
                lo = last_cnt - rem
                pltpu.async_copy(
                    table_sp.at[idx_v.at[pl.ds(lo, rem)]], rem_buf,
                    gsems[0]).wait()
                pltpu.sync_copy(
                    rem_buf, out_hbm.at[pl.ds(n_full * _W, rem)])

    return gather_kernel(table, idx)
